# baseline (speedup 1.0000x reference)
import functools

import jax
import jax.numpy as jnp
from jax import lax
from jax.experimental import pallas as pl
from jax.experimental.pallas import tpu as pltpu
from jax.experimental.pallas import tpu_sc as plsc

N_USERS = 30000
N_ITEMS = 20000
N = N_USERS + N_ITEMS
D = 64
Q = 16
NUM_LAYERS = 3
E = 800000

NC = 2
NS = 16

NP = 51200
RPT = NP // NS
NSUB = 8
SUB = RPT // NSUB

CHUNK = 128
NCHUNK = 396
EPT = CHUNK * NCHUNK
EP = EPT * NS
NBUF = 6

_MAGIC = 0x5F3759DF


def _gcn_body(ep_hbm, emb_hbm,
              x_hbm, z_hbm,
              dis_v, ed_v,
              dst0, dst1, dst2, dst3, dst4, dst5,
              s20, s21, s22, s23, s24, s25,
              rows0, rows1, rows2, rows3, rows4, rows5,
              ones_v, zbuf, zz_v, av,
              acc_sh,
              sem0, sem1, sem2, sem3, sem4, sem5):
    c = lax.axis_index("c")
    s = lax.axis_index("s")
    zeros16 = jnp.zeros((16,), jnp.float32)
    ones16 = jnp.ones((16,), jnp.float32)
    r0 = s * RPT
    b0 = s * NCHUNK
    ROWS = (rows0, rows1, rows2, rows3, rows4, rows5)
    DSTS = (dst0, dst1, dst2, dst3, dst4, dst5)
    S2S = (s20, s21, s22, s23, s24, s25)
    SEMS = (sem0, sem1, sem2, sem3, sem4, sem5)

    def _z2(i, _):
        zbuf[i, pl.ds(0, 16)] = zeros16
        return 0
    lax.fori_loop(0, SUB, _z2, 0)

    def _z3(i, _):
        ones_v[i, pl.ds(0, 16)] = ones16
        return 0
    lax.fori_loop(0, CHUNK, _z3, 0)

    def _zero_acc(jc, _):
        pltpu.sync_copy(zbuf, acc_sh.at[pl.ds(r0 + jc * SUB, SUB)])
        return 0
    lax.fori_loop(0, NSUB, _zero_acc, 0)
    plsc.subcore_barrier()

    def _deg4(i, _):
        for b in range(4):
            k = 4 * i + b
            dstv, sem = DSTS[b], SEMS[b]

            @pl.when(k >= 4)
            def _dwait(dstv=dstv, sem=sem):
                pltpu.make_async_copy(
                    ones_v, acc_sh.at[dstv], sem).wait()

            blk = (b0 + k) * (2 * CHUNK) + CHUNK
            pltpu.sync_copy(ep_hbm.at[pl.ds(blk, CHUNK)], dstv)
            pltpu.async_copy(ones_v, acc_sh.at[dstv], sem, add=True)
        return 0
    lax.fori_loop(0, NCHUNK // 4, _deg4, 0)
    for b in range(4):
        pltpu.make_async_copy(ones_v, acc_sh.at[DSTS[b]], SEMS[b]).wait()
    plsc.subcore_barrier()

    half16 = jnp.full((16,), 0.5, jnp.float32)
    magic16 = jnp.full((16,), _MAGIC, jnp.int32)
    iota16 = lax.iota(jnp.int32, 16)
    zeroi16 = jnp.zeros((16,), jnp.int32)
    def _disj(jc, _):
        rb = r0 + jc * SUB
        pltpu.sync_copy(acc_sh.at[pl.ds(rb, SUB)], av)
        def _g(g, _):
            ridx = lax.broadcast_in_dim(g * 16, (16,), ()) + iota16
            d = plsc.load_gather(av, [ridx, zeroi16])
            bits = plsc.bitcast(d, jnp.int32)
            bits = magic16 - lax.shift_right_logical(bits, 1)
            y = plsc.bitcast(bits, jnp.float32)
            for _n in range(3):
                y = y * (1.5 - 0.5 * d * y * y)
            dis_v[pl.ds(jc * SUB + g * 16, 16)] = jnp.where(
                d > half16, y, zeros16)
            return 0
        lax.fori_loop(0, SUB // 16, _g, 0)
        return 0
    lax.fori_loop(0, NSUB, _disj, 0)

    def _layer(l, __):
        lgt0 = l > 0
        def _pass(p, __p):
            qoff = (2 * c + p) * NP
            q16 = lax.broadcast_in_dim(qoff, (16,), ())

            def _prep(k, bi):
                rows, dstv, s2v, sem = ROWS[bi], DSTS[bi], S2S[bi], SEMS[bi]
                blk = (b0 + k) * (2 * CHUNK)
                pltpu.sync_copy(ep_hbm.at[pl.ds(blk, 2 * CHUNK)], ed_v)
                for j in range(CHUNK // 16):
                    sl = pl.ds(j * 16, 16)
                    s2v[sl] = ed_v[sl] + q16
                    dstv[sl] = ed_v[pl.ds(CHUNK + j * 16, 16)]
                pltpu.async_copy(z_hbm.at[s2v], rows, sem)

            @pl.when(lgt0)
            def _edge_phase():
                def _zero(jc, _):
                    pltpu.sync_copy(zbuf, acc_sh.at[pl.ds(r0 + jc * SUB, SUB)])
                    return 0
                lax.fori_loop(0, NSUB, _zero, 0)
                plsc.subcore_barrier()

                _prep(0, 0)
                _prep(1, 1)

                def _six(i, _):
                    for b in range(NBUF):
                        k = 6 * i + b
                        bn = (b + 2) % NBUF
                        rows, dstv, s2v, sem = (
                            ROWS[b], DSTS[b], S2S[b], SEMS[b])
                        rn, dn, semn = ROWS[bn], DSTS[bn], SEMS[bn]

                        @pl.when(k >= 4)
                        def _wait_s(rn=rn, dn=dn, semn=semn):
                            pltpu.make_async_copy(
                                rn, acc_sh.at[dn], semn).wait()

                        @pl.when(k <= NCHUNK - 3)
                        def _next(k=k, bn=bn):
                            _prep(k + 2, bn)

                        pltpu.make_async_copy(
                            z_hbm.at[s2v], rows, sem).wait()
                        pltpu.async_copy(
                            rows, acc_sh.at[dstv], sem, add=True)
                    return 0
                lax.fori_loop(0, NCHUNK // NBUF, _six, 0)
                for m in range(NCHUNK - 4, NCHUNK):
                    b = m % NBUF
                    pltpu.make_async_copy(
                        ROWS[b], acc_sh.at[DSTS[b]], SEMS[b]).wait()
                plsc.subcore_barrier()

            m16 = jnp.where(
                lax.broadcast_in_dim(lgt0, (16,), ()), ones16, zeros16)

            def _epi(jc, _):
                rb = r0 + jc * SUB
                gb = qoff + rb

                @pl.when(lgt0)
                def _from_acc():
                    pltpu.sync_copy(acc_sh.at[pl.ds(rb, SUB)], av)

                @pl.when(jnp.logical_not(lgt0))
                def _from_emb():
                    pltpu.sync_copy(emb_hbm.at[pl.ds(gb, SUB)], av)

                def _row(rr, _):
                    ridx = lax.broadcast_in_dim(jc * SUB + rr, (16,), ())
                    dv = plsc.load_gather(dis_v, [ridx])
                    scale = dv * m16 + (ones16 - m16)
                    sl = pl.ds(0, 16)
                    x = av[rr, sl] * scale
                    av[rr, sl] = x
                    zz_v[rr, sl] = x * dv
                    return 0
                lax.fori_loop(0, SUB, _row, 0, unroll=4)
                pltpu.sync_copy(av, x_hbm.at[pl.ds(l * 4 * NP + gb, SUB)])
                @pl.when(l < NUM_LAYERS)
                def _zw():
                    pltpu.sync_copy(zz_v, z_hbm.at[pl.ds(gb, SUB)])
                return 0
            lax.fori_loop(0, NSUB, _epi, 0)
            plsc.subcore_barrier()
            return 0
        lax.fori_loop(0, 2, _pass, 0)
        return 0

    lax.fori_loop(0, NUM_LAYERS + 1, _layer, 0)


def _mean_body(a_ref, b_ref, c_ref, d_ref, o_ref):
    o_ref[...] = 0.25 * (a_ref[...] + b_ref[...] + c_ref[...] + d_ref[...])


@functools.partial(jax.jit, static_argnums=())
def _gcn(ep, emb_cat):
    mesh = plsc.VectorSubcoreMesh(
        core_axis_name="c", subcore_axis_name="s",
        num_cores=NC, num_subcores=NS)
    f = pl.kernel(
        _gcn_body,
        out_type=[
            pltpu.HBM(((NUM_LAYERS + 1) * 4 * NP, Q), jnp.float32),
            pltpu.HBM((4 * NP, Q), jnp.float32),
        ],
        mesh=mesh,
        compiler_params=pltpu.CompilerParams(
            needs_layout_passes=False, use_tc_tiling_on_sc=False),
        scratch_types=(
            [pltpu.VMEM((RPT,), jnp.float32),
             pltpu.VMEM((2 * CHUNK,), jnp.int32)]
            + [pltpu.VMEM((CHUNK,), jnp.int32) for _ in range(NBUF)]
            + [pltpu.VMEM((CHUNK,), jnp.int32) for _ in range(NBUF)]
            + [pltpu.VMEM((CHUNK, Q), jnp.float32) for _ in range(NBUF)]
            + [pltpu.VMEM((CHUNK, Q), jnp.float32),
               pltpu.VMEM((SUB, Q), jnp.float32),
               pltpu.VMEM((SUB, Q), jnp.float32),
               pltpu.VMEM((SUB, Q), jnp.float32),
               pltpu.VMEM_SHARED((NP, Q), jnp.float32)]
            + [pltpu.SemaphoreType.DMA for _ in range(NBUF)]
        ),
    )
    x_all, _z = f(ep, emb_cat)

    ROWS = 4 * NP * Q // 128
    BR = 256
    xf = x_all.reshape((NUM_LAYERS + 1) * ROWS, 128)
    spec_l = lambda l: pl.BlockSpec(
        (BR, 128), lambda i, l=l: (l * (ROWS // BR) + i, 0))
    mean_flat = pl.pallas_call(
        _mean_body,
        grid=(ROWS // BR,),
        in_specs=[spec_l(0), spec_l(1), spec_l(2), spec_l(3)],
        out_specs=pl.BlockSpec((BR, 128), lambda i: (i, 0)),
        out_shape=jax.ShapeDtypeStruct((ROWS, 128), jnp.float32),
    )(xf, xf, xf, xf)
    return mean_flat.reshape(4 * NP, Q)


def kernel(edge_index, edge_attrs, embedding):
    del edge_attrs
    src = edge_index[0].astype(jnp.int32)
    dst = edge_index[1].astype(jnp.int32)
    pad = jnp.full((EP - E,), N, jnp.int32)
    src_pad = jnp.concatenate([src, pad]).reshape(-1, CHUNK)
    dst_pad = jnp.concatenate([dst, pad]).reshape(-1, CHUNK)
    ep = jnp.stack([src_pad, dst_pad], axis=1).reshape(-1)
    emb_pad = jnp.zeros((NP, D), jnp.float32).at[:N].set(embedding)
    emb_cat = jnp.concatenate(
        [emb_pad[:, i * Q:(i + 1) * Q] for i in range(4)], axis=0)
    m = _gcn(ep, emb_cat)
    return jnp.concatenate(
        [m[i * NP:i * NP + N] for i in range(4)], axis=1)


# speedup vs baseline: 1.0024x; 1.0024x over previous
import functools

import jax
import jax.numpy as jnp
from jax import lax
from jax.experimental import pallas as pl
from jax.experimental.pallas import tpu as pltpu
from jax.experimental.pallas import tpu_sc as plsc

N_USERS = 30000
N_ITEMS = 20000
N = N_USERS + N_ITEMS
D = 64
Q = 16
NUM_LAYERS = 3
E = 800000

NC = 2
NS = 16

NP = 51200
RPT = NP // NS
NSUB = 8
SUB = RPT // NSUB

CHUNK = 128
NCHUNK = 396
EPT = CHUNK * NCHUNK
EP = EPT * NS
NBUF = 6

_MAGIC = 0x5F3759DF


def _gcn_body(ep_hbm, emb_hbm,
              x_hbm, z_hbm,
              dis_v, ed_v,
              dst0, dst1, dst2, dst3, dst4, dst5,
              s20, s21, s22, s23, s24, s25,
              rows0, rows1, rows2, rows3, rows4, rows5,
              ones_v, zbuf, zz_v, av,
              acc_sh,
              sem0, sem1, sem2, sem3, sem4, sem5):
    c = lax.axis_index("c")
    s = lax.axis_index("s")
    zeros16 = jnp.zeros((16,), jnp.float32)
    ones16 = jnp.ones((16,), jnp.float32)
    r0 = s * RPT
    b0 = s * NCHUNK
    ROWS = (rows0, rows1, rows2, rows3, rows4, rows5)
    DSTS = (dst0, dst1, dst2, dst3, dst4, dst5)
    S2S = (s20, s21, s22, s23, s24, s25)
    SEMS = (sem0, sem1, sem2, sem3, sem4, sem5)

    def _z2(i, _):
        zbuf[i, pl.ds(0, 16)] = zeros16
        return 0
    lax.fori_loop(0, SUB, _z2, 0)

    def _z3(i, _):
        ones_v[i, pl.ds(0, 16)] = ones16
        return 0
    lax.fori_loop(0, CHUNK, _z3, 0)

    def _zero_acc(jc, _):
        pltpu.sync_copy(zbuf, acc_sh.at[pl.ds(r0 + jc * SUB, SUB)])
        return 0
    lax.fori_loop(0, NSUB, _zero_acc, 0)
    plsc.subcore_barrier()

    def _deg4(i, _):
        for b in range(4):
            k = 4 * i + b
            dstv, sem = DSTS[b], SEMS[b]

            @pl.when(k >= 4)
            def _dwait(dstv=dstv, sem=sem):
                pltpu.make_async_copy(
                    ones_v, acc_sh.at[dstv], sem).wait()

            blk = (b0 + k) * (2 * CHUNK) + CHUNK
            pltpu.sync_copy(ep_hbm.at[pl.ds(blk, CHUNK)], dstv)
            pltpu.async_copy(ones_v, acc_sh.at[dstv], sem, add=True)
        return 0
    lax.fori_loop(0, NCHUNK // 4, _deg4, 0)
    for b in range(4):
        pltpu.make_async_copy(ones_v, acc_sh.at[DSTS[b]], SEMS[b]).wait()
    plsc.subcore_barrier()

    half16 = jnp.full((16,), 0.5, jnp.float32)
    magic16 = jnp.full((16,), _MAGIC, jnp.int32)
    iota16 = lax.iota(jnp.int32, 16)
    zeroi16 = jnp.zeros((16,), jnp.int32)
    def _disj(jc, _):
        rb = r0 + jc * SUB
        pltpu.sync_copy(acc_sh.at[pl.ds(rb, SUB)], av)
        def _g(g, _):
            ridx = lax.broadcast_in_dim(g * 16, (16,), ()) + iota16
            d = plsc.load_gather(av, [ridx, zeroi16])
            bits = plsc.bitcast(d, jnp.int32)
            bits = magic16 - lax.shift_right_logical(bits, 1)
            y = plsc.bitcast(bits, jnp.float32)
            for _n in range(3):
                y = y * (1.5 - 0.5 * d * y * y)
            dis_v[pl.ds(jc * SUB + g * 16, 16)] = jnp.where(
                d > half16, y, zeros16)
            return 0
        lax.fori_loop(0, SUB // 16, _g, 0)
        return 0
    lax.fori_loop(0, NSUB, _disj, 0)

    def _layer(l, __):
        lgt0 = l > 0
        def _pass(p, __p):
            qoff = (2 * c + p) * NP
            q16 = lax.broadcast_in_dim(qoff, (16,), ())

            def _prep(k, bi):
                rows, dstv, s2v, sem = ROWS[bi], DSTS[bi], S2S[bi], SEMS[bi]
                blk = (b0 + k) * (2 * CHUNK)
                pltpu.sync_copy(ep_hbm.at[pl.ds(blk, 2 * CHUNK)], ed_v)
                for j in range(CHUNK // 16):
                    sl = pl.ds(j * 16, 16)
                    s2v[sl] = ed_v[sl] + q16
                    dstv[sl] = ed_v[pl.ds(CHUNK + j * 16, 16)]
                pltpu.async_copy(z_hbm.at[s2v], rows, sem)

            @pl.when(lgt0)
            def _edge_phase():
                def _zero(jc, _):
                    pltpu.sync_copy(zbuf, acc_sh.at[pl.ds(r0 + jc * SUB, SUB)])
                    return 0
                lax.fori_loop(0, NSUB, _zero, 0)
                plsc.subcore_barrier()

                _prep(0, 0)
                _prep(1, 1)

                def _six(i, _):
                    for b in range(NBUF):
                        k = 6 * i + b
                        bn = (b + 2) % NBUF
                        rows, dstv, s2v, sem = (
                            ROWS[b], DSTS[b], S2S[b], SEMS[b])
                        rn, dn, semn = ROWS[bn], DSTS[bn], SEMS[bn]

                        @pl.when(k >= 4)
                        def _wait_s(rn=rn, dn=dn, semn=semn):
                            pltpu.make_async_copy(
                                rn, acc_sh.at[dn], semn).wait()

                        @pl.when(k <= NCHUNK - 3)
                        def _next(k=k, bn=bn):
                            _prep(k + 2, bn)

                        pltpu.make_async_copy(
                            z_hbm.at[s2v], rows, sem).wait()
                        pltpu.async_copy(
                            rows, acc_sh.at[dstv], sem, add=True)
                    return 0
                lax.fori_loop(0, NCHUNK // NBUF, _six, 0)
                for m in range(NCHUNK - 4, NCHUNK):
                    b = m % NBUF
                    pltpu.make_async_copy(
                        ROWS[b], acc_sh.at[DSTS[b]], SEMS[b]).wait()
                plsc.subcore_barrier()

            m16 = jnp.where(
                lax.broadcast_in_dim(lgt0, (16,), ()), ones16, zeros16)

            def _epi(jc, _):
                rb = r0 + jc * SUB
                gb = qoff + rb

                @pl.when(lgt0)
                def _from_acc():
                    pltpu.sync_copy(acc_sh.at[pl.ds(rb, SUB)], av)

                @pl.when(jnp.logical_not(lgt0))
                def _from_emb():
                    pltpu.sync_copy(emb_hbm.at[pl.ds(gb, SUB)], av)

                def _row(rr, _):
                    ridx = lax.broadcast_in_dim(jc * SUB + rr, (16,), ())
                    dv = plsc.load_gather(dis_v, [ridx])
                    scale = dv * m16 + (ones16 - m16)
                    sl = pl.ds(0, 16)
                    x = av[rr, sl] * scale
                    av[rr, sl] = x
                    zz_v[rr, sl] = x * dv
                    return 0
                lax.fori_loop(0, SUB, _row, 0)
                pltpu.sync_copy(av, x_hbm.at[pl.ds(l * 4 * NP + gb, SUB)])
                @pl.when(l < NUM_LAYERS)
                def _zw():
                    pltpu.sync_copy(zz_v, z_hbm.at[pl.ds(gb, SUB)])
                return 0
            lax.fori_loop(0, NSUB, _epi, 0)
            plsc.subcore_barrier()
            return 0
        lax.fori_loop(0, 2, _pass, 0)
        return 0

    lax.fori_loop(0, NUM_LAYERS + 1, _layer, 0)


def _mean_body(a_ref, b_ref, c_ref, d_ref, o_ref):
    o_ref[...] = 0.25 * (a_ref[...] + b_ref[...] + c_ref[...] + d_ref[...])


@functools.partial(jax.jit, static_argnums=())
def _gcn(ep, emb_cat):
    mesh = plsc.VectorSubcoreMesh(
        core_axis_name="c", subcore_axis_name="s",
        num_cores=NC, num_subcores=NS)
    f = pl.kernel(
        _gcn_body,
        out_type=[
            pltpu.HBM(((NUM_LAYERS + 1) * 4 * NP, Q), jnp.float32),
            pltpu.HBM((4 * NP, Q), jnp.float32),
        ],
        mesh=mesh,
        compiler_params=pltpu.CompilerParams(
            needs_layout_passes=False, use_tc_tiling_on_sc=False),
        scratch_types=(
            [pltpu.VMEM((RPT,), jnp.float32),
             pltpu.VMEM((2 * CHUNK,), jnp.int32)]
            + [pltpu.VMEM((CHUNK,), jnp.int32) for _ in range(NBUF)]
            + [pltpu.VMEM((CHUNK,), jnp.int32) for _ in range(NBUF)]
            + [pltpu.VMEM((CHUNK, Q), jnp.float32) for _ in range(NBUF)]
            + [pltpu.VMEM((CHUNK, Q), jnp.float32),
               pltpu.VMEM((SUB, Q), jnp.float32),
               pltpu.VMEM((SUB, Q), jnp.float32),
               pltpu.VMEM((SUB, Q), jnp.float32),
               pltpu.VMEM_SHARED((NP, Q), jnp.float32)]
            + [pltpu.SemaphoreType.DMA for _ in range(NBUF)]
        ),
    )
    x_all, _z = f(ep, emb_cat)

    ROWS = 4 * NP * Q // 128
    BR = 256
    xf = x_all.reshape((NUM_LAYERS + 1) * ROWS, 128)
    spec_l = lambda l: pl.BlockSpec(
        (BR, 128), lambda i, l=l: (l * (ROWS // BR) + i, 0))
    mean_flat = pl.pallas_call(
        _mean_body,
        grid=(ROWS // BR,),
        in_specs=[spec_l(0), spec_l(1), spec_l(2), spec_l(3)],
        out_specs=pl.BlockSpec((BR, 128), lambda i: (i, 0)),
        out_shape=jax.ShapeDtypeStruct((ROWS, 128), jnp.float32),
    )(xf, xf, xf, xf)
    return mean_flat.reshape(4 * NP, Q)


def kernel(edge_index, edge_attrs, embedding):
    del edge_attrs
    src = edge_index[0].astype(jnp.int32)
    dst = edge_index[1].astype(jnp.int32)
    pad = jnp.full((EP - E,), N, jnp.int32)
    src_pad = jnp.concatenate([src, pad]).reshape(-1, CHUNK)
    dst_pad = jnp.concatenate([dst, pad]).reshape(-1, CHUNK)
    ep = jnp.stack([src_pad, dst_pad], axis=1).reshape(-1)
    emb_pad = jnp.zeros((NP, D), jnp.float32).at[:N].set(embedding)
    emb_cat = jnp.concatenate(
        [emb_pad[:, i * Q:(i + 1) * Q] for i in range(4)], axis=0)
    m = _gcn(ep, emb_cat)
    return jnp.concatenate(
        [m[i * NP:i * NP + N] for i in range(4)], axis=1)
